# baseline (speedup 1.0000x reference)
import functools

import jax
import jax.numpy as jnp
from jax import lax
from jax.experimental import pallas as pl
from jax.experimental.pallas import tpu as pltpu
from jax.experimental.pallas import tpu_sc as plsc

FEAT = 512
BATCH = 16384
NC, NS, L = 2, 16, 16
NW = NC * NS
BPW = BATCH // NW
C = 32
NCHUNK = BPW // C
NPAIR = NCHUNK // 2
NACC = 4


def _sc_body(mu_hbm, lv_hbm, lab_hbm, fcw_hbm, out_hbm,
             idx_v, g0, m0, l0, g1, m1, l1, acc_v, sem0, sem1):
    wid = lax.axis_index("s") * NC + lax.axis_index("c")
    base = wid * BPW
    pltpu.sync_copy(lab_hbm.at[pl.ds(base, BPW)], idx_v)

    def issue(k, g, m, l, sem):
        row0 = base + k * C
        pltpu.async_copy(fcw_hbm.at[idx_v.at[pl.ds(k * C, C)]], g, sem)
        pltpu.async_copy(mu_hbm.at[pl.ds(row0, C)], m, sem)
        pltpu.async_copy(lv_hbm.at[pl.ds(row0, C)], l, sem)

    def drain(k, g, m, l, sem):
        row0 = base + k * C
        pltpu.make_async_copy(fcw_hbm.at[idx_v.at[pl.ds(k * C, C)]], g, sem).wait()
        pltpu.make_async_copy(mu_hbm.at[pl.ds(row0, C)], m, sem).wait()
        pltpu.make_async_copy(lv_hbm.at[pl.ds(row0, C)], l, sem).wait()

    def consume(g_v, mu_v, lv_v, acc):
        def row(r, acc):
            af = list(acc[0])
            av = list(acc[1])
            for c in range(FEAT // L):
                sl = pl.ds(c * L, L)
                g = g_v[r, sl]
                m = mu_v[r, sl]
                v = lv_v[r, sl]
                d = g - m
                j = c % NACC
                af[j] = af[j] + (d * d) * jnp.exp(-v)
                av[j] = av[j] + v
            return tuple(af), tuple(av)

        return lax.fori_loop(0, C, row, acc)

    issue(0, g0, m0, l0, sem0)
    issue(1, g1, m1, l1, sem1)

    def pair(p, acc):
        k0 = 2 * p
        drain(k0, g0, m0, l0, sem0)
        acc = consume(g0, m0, l0, acc)
        issue(jnp.minimum(k0 + 2, NCHUNK - 1), g0, m0, l0, sem0)
        drain(k0 + 1, g1, m1, l1, sem1)
        acc = consume(g1, m1, l1, acc)
        issue(jnp.minimum(k0 + 3, NCHUNK - 1), g1, m1, l1, sem1)
        return acc

    zero = jnp.zeros((L,), jnp.float32)
    init = (tuple(zero for _ in range(NACC)), tuple(zero for _ in range(NACC)))
    af, av = lax.fori_loop(0, NPAIR, pair, init)
    drain(NCHUNK - 1, g0, m0, l0, sem0)
    drain(NCHUNK - 1, g1, m1, l1, sem1)

    tot = af[0] + av[0]
    for j in range(1, NACC):
        tot = tot + af[j] + av[j]
    acc_v[...] = tot
    pltpu.sync_copy(acc_v, out_hbm.at[wid])


def kernel(mu, logvar, labels, fc_weights):
    labels = labels.astype(jnp.int32)
    mesh = plsc.VectorSubcoreMesh(
        core_axis_name="c", subcore_axis_name="s",
        num_cores=NC, num_subcores=NS)
    buf = lambda: pltpu.VMEM((C, FEAT), jnp.float32)
    partials = pl.kernel(
        _sc_body,
        out_type=jax.ShapeDtypeStruct((NW, L), jnp.float32),
        mesh=mesh,
        scratch_types=[
            pltpu.VMEM((BPW,), jnp.int32),
            buf(), buf(), buf(), buf(), buf(), buf(),
            pltpu.VMEM((L,), jnp.float32),
            pltpu.SemaphoreType.DMA,
            pltpu.SemaphoreType.DMA,
        ],
    )(mu, logvar, labels, fc_weights)
    return jnp.sum(partials) / (2.0 * BATCH)


# speedup vs baseline: 1.1186x; 1.1186x over previous
import functools

import jax
import jax.numpy as jnp
from jax import lax
from jax.experimental import pallas as pl
from jax.experimental.pallas import tpu as pltpu
from jax.experimental.pallas import tpu_sc as plsc

FEAT = 512
BATCH = 16384
NC, NS, L = 2, 16, 16
NW = NC * NS
BPW = BATCH // NW
C = 32
NCHUNK = BPW // C
NPAIR = NCHUNK // 2
NACC = 4


def _sc_body(mu_hbm, lv_hbm, lab_hbm, fcw_hbm, out_hbm,
             idx_v, g0, m0, l0, g1, m1, l1, acc_v, sem0, sem1):
    wid = lax.axis_index("s") * NC + lax.axis_index("c")
    base = wid * BPW
    pltpu.sync_copy(lab_hbm.at[pl.ds(base, BPW)], idx_v)

    def issue(k, g, m, l, sem):
        row0 = base + k * C
        pltpu.async_copy(fcw_hbm.at[idx_v.at[pl.ds(k * C, C)]], g, sem)
        pltpu.async_copy(mu_hbm.at[pl.ds(row0, C)], m, sem)
        pltpu.async_copy(lv_hbm.at[pl.ds(row0, C)], l, sem)

    def drain(k, g, m, l, sem):
        row0 = base + k * C
        pltpu.make_async_copy(fcw_hbm.at[idx_v.at[pl.ds(k * C, C)]], g, sem).wait()
        pltpu.make_async_copy(mu_hbm.at[pl.ds(row0, C)], m, sem).wait()
        pltpu.make_async_copy(lv_hbm.at[pl.ds(row0, C)], l, sem).wait()

    def consume(g_v, mu_v, lv_v, acc):
        def row(r, acc):
            af, av = acc
            sf = [None] * NACC
            sv = [None] * NACC
            for c in range(FEAT // L):
                sl = pl.ds(c * L, L)
                g = g_v[r, sl]
                m = mu_v[r, sl]
                v = lv_v[r, sl]
                d = g - m
                t = (d * d) * jnp.exp(-v)
                j = c % NACC
                sf[j] = t if sf[j] is None else sf[j] + t
                sv[j] = v if sv[j] is None else sv[j] + v
            af = af + ((sf[0] + sf[1]) + (sf[2] + sf[3]))
            av = av + ((sv[0] + sv[1]) + (sv[2] + sv[3]))
            return af, av

        return lax.fori_loop(0, C, row, acc)

    issue(0, g0, m0, l0, sem0)
    issue(1, g1, m1, l1, sem1)

    def pair(p, acc):
        k0 = 2 * p
        drain(k0, g0, m0, l0, sem0)
        acc = consume(g0, m0, l0, acc)
        issue(jnp.minimum(k0 + 2, NCHUNK - 1), g0, m0, l0, sem0)
        drain(k0 + 1, g1, m1, l1, sem1)
        acc = consume(g1, m1, l1, acc)
        issue(jnp.minimum(k0 + 3, NCHUNK - 1), g1, m1, l1, sem1)
        return acc

    zero = jnp.zeros((L,), jnp.float32)
    af, av = lax.fori_loop(0, NPAIR, pair, (zero, zero))
    drain(NCHUNK - 1, g0, m0, l0, sem0)
    drain(NCHUNK - 1, g1, m1, l1, sem1)

    acc_v[...] = af + av
    pltpu.sync_copy(acc_v, out_hbm.at[wid])


def kernel(mu, logvar, labels, fc_weights):
    labels = labels.astype(jnp.int32)
    mesh = plsc.VectorSubcoreMesh(
        core_axis_name="c", subcore_axis_name="s",
        num_cores=NC, num_subcores=NS)
    buf = lambda: pltpu.VMEM((C, FEAT), jnp.float32)
    partials = pl.kernel(
        _sc_body,
        out_type=jax.ShapeDtypeStruct((NW, L), jnp.float32),
        mesh=mesh,
        scratch_types=[
            pltpu.VMEM((BPW,), jnp.int32),
            buf(), buf(), buf(), buf(), buf(), buf(),
            pltpu.VMEM((L,), jnp.float32),
            pltpu.SemaphoreType.DMA,
            pltpu.SemaphoreType.DMA,
        ],
    )(mu, logvar, labels, fc_weights)
    return jnp.sum(partials) / (2.0 * BATCH)


# speedup vs baseline: 1.2219x; 1.0924x over previous
import functools

import jax
import jax.numpy as jnp
from jax import lax
from jax.experimental import pallas as pl
from jax.experimental.pallas import tpu as pltpu
from jax.experimental.pallas import tpu_sc as plsc

FEAT = 512
BATCH = 16384
NC, NS, L = 2, 16, 16
NW = NC * NS
BPW = BATCH // NW
C = 32
NCHUNK = BPW // C
NPAIR = NCHUNK // 2
NACC = 4


def _sc_body(mu_hbm, lv_hbm, lab_hbm, fcw_hbm, out_hbm,
             idx_v, g0, m0, l0, g1, m1, l1, acc_v, sem0, sem1):
    wid = lax.axis_index("s") * NC + lax.axis_index("c")
    base = wid * BPW
    pltpu.sync_copy(lab_hbm.at[pl.ds(base, BPW)], idx_v)

    def issue(k, g, m, l, sem):
        row0 = base + k * C
        pltpu.async_copy(fcw_hbm.at[idx_v.at[pl.ds(k * C, C)]], g, sem)
        pltpu.async_copy(mu_hbm.at[pl.ds(row0, C)], m, sem)
        pltpu.async_copy(lv_hbm.at[pl.ds(row0, C)], l, sem)

    def drain(k, g, m, l, sem):
        row0 = base + k * C
        pltpu.make_async_copy(fcw_hbm.at[idx_v.at[pl.ds(k * C, C)]], g, sem).wait()
        pltpu.make_async_copy(mu_hbm.at[pl.ds(row0, C)], m, sem).wait()
        pltpu.make_async_copy(lv_hbm.at[pl.ds(row0, C)], l, sem).wait()

    def consume(g_v, mu_v, lv_v, acc):
        def row(r, acc):
            af, av = acc
            for c in range(FEAT // L):
                sl = pl.ds(c * L, L)
                g = g_v[r, sl]
                m = mu_v[r, sl]
                v = lv_v[r, sl]
                d = g - m
                af = af + (d * d) * jnp.exp(-v)
                av = av + v
            return af, av

        return lax.fori_loop(0, C, row, acc)

    issue(0, g0, m0, l0, sem0)
    issue(1, g1, m1, l1, sem1)

    def pair(p, acc):
        k0 = 2 * p
        drain(k0, g0, m0, l0, sem0)
        acc = consume(g0, m0, l0, acc)
        issue(jnp.minimum(k0 + 2, NCHUNK - 1), g0, m0, l0, sem0)
        drain(k0 + 1, g1, m1, l1, sem1)
        acc = consume(g1, m1, l1, acc)
        issue(jnp.minimum(k0 + 3, NCHUNK - 1), g1, m1, l1, sem1)
        return acc

    zero = jnp.zeros((L,), jnp.float32)
    af, av = lax.fori_loop(0, NPAIR, pair, (zero, zero))
    drain(NCHUNK - 1, g0, m0, l0, sem0)
    drain(NCHUNK - 1, g1, m1, l1, sem1)

    acc_v[...] = af + av
    pltpu.sync_copy(acc_v, out_hbm.at[wid])


def kernel(mu, logvar, labels, fc_weights):
    labels = labels.astype(jnp.int32)
    mesh = plsc.VectorSubcoreMesh(
        core_axis_name="c", subcore_axis_name="s",
        num_cores=NC, num_subcores=NS)
    buf = lambda: pltpu.VMEM((C, FEAT), jnp.float32)
    partials = pl.kernel(
        _sc_body,
        out_type=jax.ShapeDtypeStruct((NW, L), jnp.float32),
        mesh=mesh,
        scratch_types=[
            pltpu.VMEM((BPW,), jnp.int32),
            buf(), buf(), buf(), buf(), buf(), buf(),
            pltpu.VMEM((L,), jnp.float32),
            pltpu.SemaphoreType.DMA,
            pltpu.SemaphoreType.DMA,
        ],
    )(mu, logvar, labels, fc_weights)
    return jnp.sum(partials) / (2.0 * BATCH)


# speedup vs baseline: 1.2780x; 1.0458x over previous
import functools

import jax
import jax.numpy as jnp
from jax import lax
from jax.experimental import pallas as pl
from jax.experimental.pallas import tpu as pltpu
from jax.experimental.pallas import tpu_sc as plsc

FEAT = 512
BATCH = 16384
NC, NS, L = 2, 16, 16
NW = NC * NS
BPW = BATCH // NW
C = 32
NCHUNK = BPW // C
NPAIR = NCHUNK // 2
NACC = 4


def _sc_body(mu_hbm, lv_hbm, lab_hbm, fcw_hbm, out_hbm,
             idx_v, g0, m0, l0, g1, m1, l1, acc_v, sem0, sem1):
    wid = lax.axis_index("s") * NC + lax.axis_index("c")
    base = wid * BPW
    pltpu.sync_copy(lab_hbm.at[pl.ds(base, BPW)], idx_v)

    def issue(k, g, m, l, sem):
        row0 = base + k * C
        pltpu.async_copy(fcw_hbm.at[idx_v.at[pl.ds(k * C, C)]], g, sem)
        pltpu.async_copy(mu_hbm.at[pl.ds(row0, C)], m, sem)
        pltpu.async_copy(lv_hbm.at[pl.ds(row0, C)], l, sem)

    def drain(k, g, m, l, sem):
        row0 = base + k * C
        pltpu.make_async_copy(fcw_hbm.at[idx_v.at[pl.ds(k * C, C)]], g, sem).wait()
        pltpu.make_async_copy(mu_hbm.at[pl.ds(row0, C)], m, sem).wait()
        pltpu.make_async_copy(lv_hbm.at[pl.ds(row0, C)], l, sem).wait()

    def consume(g_v, mu_v, lv_v, acc):
        def row(r, acc):
            af, av = acc
            for c in range(FEAT // L):
                sl = pl.ds(c * L, L)
                g = g_v[r, sl]
                m = mu_v[r, sl]
                v = lv_v[r, sl]
                d = g - m
                af = af + (d * d) * jnp.exp(-v)
                av = av + v
            return af, av

        return lax.fori_loop(0, C, row, acc)

    issue(0, g0, m0, l0, sem0)
    issue(1, g1, m1, l1, sem1)

    def pair(p, acc):
        k0 = 2 * p
        not_last = p < NPAIR - 1
        drain(k0, g0, m0, l0, sem0)
        acc = consume(g0, m0, l0, acc)

        @pl.when(not_last)
        def _():
            issue(k0 + 2, g0, m0, l0, sem0)

        drain(k0 + 1, g1, m1, l1, sem1)
        acc = consume(g1, m1, l1, acc)

        @pl.when(not_last)
        def _():
            issue(k0 + 3, g1, m1, l1, sem1)

        return acc

    zero = jnp.zeros((L,), jnp.float32)
    af, av = lax.fori_loop(0, NPAIR, pair, (zero, zero))

    acc_v[...] = af + av
    pltpu.sync_copy(acc_v, out_hbm.at[wid])


def kernel(mu, logvar, labels, fc_weights):
    labels = labels.astype(jnp.int32)
    mesh = plsc.VectorSubcoreMesh(
        core_axis_name="c", subcore_axis_name="s",
        num_cores=NC, num_subcores=NS)
    buf = lambda: pltpu.VMEM((C, FEAT), jnp.float32)
    partials = pl.kernel(
        _sc_body,
        out_type=jax.ShapeDtypeStruct((NW, L), jnp.float32),
        mesh=mesh,
        scratch_types=[
            pltpu.VMEM((BPW,), jnp.int32),
            buf(), buf(), buf(), buf(), buf(), buf(),
            pltpu.VMEM((L,), jnp.float32),
            pltpu.SemaphoreType.DMA,
            pltpu.SemaphoreType.DMA,
        ],
    )(mu, logvar, labels, fc_weights)
    return jnp.sum(partials) / (2.0 * BATCH)
